# baseline (speedup 1.0000x reference)
import functools

import jax
import jax.numpy as jnp
from jax import lax
from jax.experimental import pallas as pl
from jax.experimental.pallas import tpu as pltpu
from jax.experimental.pallas import tpu_sc as plsc

_B = 4096
_L = 200
_D = 128
_V = 6

_info = plsc.get_sparse_core_info()
_NC = _info.num_cores
_NS = _info.num_subcores
_NW = _NC * _NS
_BPW = _B // _NW

_OFFS = tuple(range(0, 192, 16)) + (184,)


def _build_block(seq_all, tab_flat, buf, i):
    def row(d, _):
        dsplat = jnp.full((16,), d, jnp.int32)
        for off in _OFFS:
            s = seq_all[i, pl.ds(off, 16)]
            idx = (s << 7) + dsplat
            buf[d, pl.ds(off, 16)] = plsc.load_gather(tab_flat, [idx])
        return 0

    lax.fori_loop(0, _D, row, 0)


def _sc_body(
    seq_hbm, tab_hbm, out_hbm, seq_all, tab_v, tab_flat, buf0, buf1, sem0, sem1
):
    wid = lax.axis_index("s") * _NC + lax.axis_index("c")
    base = wid * _BPW
    pltpu.sync_copy(tab_hbm, tab_v)
    pltpu.sync_copy(seq_hbm.at[pl.ds(base, _BPW)], seq_all)
    for v in range(_V):
        for c in range(_D // 16):
            tab_flat[pl.ds(v * _D + c * 16, 16)] = tab_v[v, pl.ds(c * 16, 16)]

    def pair(j, _):
        for k, buf, sem in ((0, buf0, sem0), (1, buf1, sem1)):
            i = 2 * j + k

            @pl.when(j > 0)
            def _():
                pltpu.make_async_copy(out_hbm.at[base], buf, sem).wait()

            _build_block(seq_all, tab_flat, buf, i)
            pltpu.async_copy(buf, out_hbm.at[base + i], sem)
        return 0

    lax.fori_loop(0, _BPW // 2, pair, 0)
    pltpu.make_async_copy(out_hbm.at[base], buf0, sem0).wait()
    pltpu.make_async_copy(out_hbm.at[base], buf1, sem1).wait()


def kernel(seq, table):
    seq = seq.astype(jnp.int32)
    mesh = plsc.VectorSubcoreMesh(core_axis_name="c", subcore_axis_name="s")
    k = functools.partial(
        pl.kernel,
        mesh=mesh,
        out_type=jax.ShapeDtypeStruct((_B, _D, _L), jnp.float32),
        scratch_types=[
            pltpu.VMEM((_BPW, _L), jnp.int32),
            pltpu.VMEM((_V, _D), jnp.float32),
            pltpu.VMEM((_V * _D,), jnp.float32),
            pltpu.VMEM((_D, _L), jnp.float32),
            pltpu.VMEM((_D, _L), jnp.float32),
            pltpu.SemaphoreType.DMA,
            pltpu.SemaphoreType.DMA,
        ],
        compiler_params=pltpu.CompilerParams(needs_layout_passes=False),
    )(_sc_body)
    return k(seq, table)


# speedup vs baseline: 2.4912x; 2.4912x over previous
import functools

import jax
import jax.numpy as jnp
from jax import lax
from jax.experimental import pallas as pl
from jax.experimental.pallas import tpu as pltpu
from jax.experimental.pallas import tpu_sc as plsc

_B = 4096
_L = 200
_D = 128
_V = 6

_info = plsc.get_sparse_core_info()
_NC = _info.num_cores
_NS = _info.num_subcores
_NW = _NC * _NS
_BPW = _B // _NW

_OFFS = tuple(range(0, 192, 16)) + (184,)


def _build_block(seq_all, tab_flat, buf, i):
    s128 = [seq_all[i, pl.ds(off, 16)] << 7 for off in _OFFS]

    @plsc.parallel_loop(0, _D, unroll=4)
    def _rows(d):
        dsplat = jnp.full((16,), d, jnp.int32)
        for c, off in enumerate(_OFFS):
            buf[d, pl.ds(off, 16)] = plsc.load_gather(tab_flat, [s128[c] + dsplat])


def _sc_body(
    seq_hbm, tab_hbm, out_hbm, seq_all, tab_v, tab_flat, buf0, buf1, sem0, sem1
):
    wid = lax.axis_index("s") * _NC + lax.axis_index("c")
    base = wid * _BPW
    pltpu.sync_copy(tab_hbm, tab_v)
    pltpu.sync_copy(seq_hbm.at[pl.ds(base, _BPW)], seq_all)
    for v in range(_V):
        for c in range(_D // 16):
            tab_flat[pl.ds(v * _D + c * 16, 16)] = tab_v[v, pl.ds(c * 16, 16)]

    def pair(j, _):
        for k, buf, sem in ((0, buf0, sem0), (1, buf1, sem1)):
            i = 2 * j + k

            @pl.when(j > 0)
            def _():
                pltpu.make_async_copy(out_hbm.at[base], buf, sem).wait()

            _build_block(seq_all, tab_flat, buf, i)
            pltpu.async_copy(buf, out_hbm.at[base + i], sem)
        return 0

    lax.fori_loop(0, _BPW // 2, pair, 0)
    pltpu.make_async_copy(out_hbm.at[base], buf0, sem0).wait()
    pltpu.make_async_copy(out_hbm.at[base], buf1, sem1).wait()


def kernel(seq, table):
    seq = seq.astype(jnp.int32)
    mesh = plsc.VectorSubcoreMesh(core_axis_name="c", subcore_axis_name="s")
    k = functools.partial(
        pl.kernel,
        mesh=mesh,
        out_type=jax.ShapeDtypeStruct((_B, _D, _L), jnp.float32),
        scratch_types=[
            pltpu.VMEM((_BPW, _L), jnp.int32),
            pltpu.VMEM((_V, _D), jnp.float32),
            pltpu.VMEM((_V * _D,), jnp.float32),
            pltpu.VMEM((_D, _L), jnp.float32),
            pltpu.VMEM((_D, _L), jnp.float32),
            pltpu.SemaphoreType.DMA,
            pltpu.SemaphoreType.DMA,
        ],
        compiler_params=pltpu.CompilerParams(needs_layout_passes=False),
    )(_sc_body)
    return k(seq, table)


# speedup vs baseline: 7.2374x; 2.9051x over previous
import functools

import jax
import jax.numpy as jnp
from jax import lax
from jax.experimental import pallas as pl
from jax.experimental.pallas import tpu as pltpu
from jax.experimental.pallas import tpu_sc as plsc

_B = 4096
_L = 200
_D = 128
_V = 6

_info = plsc.get_sparse_core_info()
_NC = _info.num_cores
_NS = _info.num_subcores
_NW = _NC * _NS
_BPW = _B // _NW

_OFFS = tuple(range(0, 192, 16)) + (184,)

_GDN = lax.GatherDimensionNumbers(
    offset_dims=(), collapsed_slice_dims=(0,), start_index_map=(0,)
)


def _vgather(lut_row, idx):
    return lax.gather(
        lut_row,
        idx[:, None],
        _GDN,
        (1,),
        mode=lax.GatherScatterMode.PROMISE_IN_BOUNDS,
    )


def _build_block(seq_all, tabt_v, buf, i):
    chunks = [seq_all[i, pl.ds(off, 16)] for off in _OFFS]

    @plsc.parallel_loop(0, _D, unroll=4)
    def _rows(d):
        lut_row = tabt_v[d, pl.ds(0, 16)]
        for c, off in enumerate(_OFFS):
            buf[d, pl.ds(off, 16)] = _vgather(lut_row, chunks[c])


def _sc_body(seq_hbm, tabt_hbm, out_hbm, seq_all, tabt_v, buf0, buf1, sem0, sem1):
    wid = lax.axis_index("s") * _NC + lax.axis_index("c")
    base = wid * _BPW
    pltpu.sync_copy(tabt_hbm, tabt_v)
    pltpu.sync_copy(seq_hbm.at[pl.ds(base, _BPW)], seq_all)

    def pair(j, _):
        for k, buf, sem in ((0, buf0, sem0), (1, buf1, sem1)):
            i = 2 * j + k

            @pl.when(j > 0)
            def _():
                pltpu.make_async_copy(out_hbm.at[base], buf, sem).wait()

            _build_block(seq_all, tabt_v, buf, i)
            pltpu.async_copy(buf, out_hbm.at[base + i], sem)
        return 0

    lax.fori_loop(0, _BPW // 2, pair, 0)
    pltpu.make_async_copy(out_hbm.at[base], buf0, sem0).wait()
    pltpu.make_async_copy(out_hbm.at[base], buf1, sem1).wait()


def kernel(seq, table):
    seq = seq.astype(jnp.int32)
    tabT16 = jnp.zeros((_D, 16), jnp.float32).at[:, :_V].set(table.T)
    mesh = plsc.VectorSubcoreMesh(core_axis_name="c", subcore_axis_name="s")
    k = functools.partial(
        pl.kernel,
        mesh=mesh,
        out_type=jax.ShapeDtypeStruct((_B, _D, _L), jnp.float32),
        scratch_types=[
            pltpu.VMEM((_BPW, _L), jnp.int32),
            pltpu.VMEM((_D, 16), jnp.float32),
            pltpu.VMEM((_D, _L), jnp.float32),
            pltpu.VMEM((_D, _L), jnp.float32),
            pltpu.SemaphoreType.DMA,
            pltpu.SemaphoreType.DMA,
        ],
        compiler_params=pltpu.CompilerParams(needs_layout_passes=False),
    )(_sc_body)
    return k(seq, tabT16)


# speedup vs baseline: 7.2381x; 1.0001x over previous
import functools

import jax
import jax.numpy as jnp
from jax import lax
from jax.experimental import pallas as pl
from jax.experimental.pallas import tpu as pltpu
from jax.experimental.pallas import tpu_sc as plsc

_B = 4096
_L = 200
_D = 128
_V = 6

_info = plsc.get_sparse_core_info()
_NC = _info.num_cores
_NS = _info.num_subcores
_NW = _NC * _NS
_BPW = _B // _NW

_OFFS = tuple(range(0, 192, 16)) + (184,)

_GDN = lax.GatherDimensionNumbers(
    offset_dims=(), collapsed_slice_dims=(0,), start_index_map=(0,)
)


def _vgather(lut_row, idx):
    return lax.gather(
        lut_row,
        idx[:, None],
        _GDN,
        (1,),
        mode=lax.GatherScatterMode.PROMISE_IN_BOUNDS,
    )


def _build_block(seq_all, tabt_v, buf, i):
    chunks = [seq_all[i, pl.ds(off, 16)] for off in _OFFS]

    @plsc.parallel_loop(0, _D, unroll=4)
    def _rows(d):
        lut_row = tabt_v[d, pl.ds(0, 16)]
        for c, off in enumerate(_OFFS):
            buf[d, pl.ds(off, 16)] = _vgather(lut_row, chunks[c])


def _sc_body(seq_hbm, tabt_hbm, out_hbm, seq_all, tabt_v, buf0, buf1, sem0, sem1):
    wid = lax.axis_index("s") * _NC + lax.axis_index("c")
    base = wid * _BPW
    pltpu.sync_copy(tabt_hbm, tabt_v)
    pltpu.sync_copy(seq_hbm.at[pl.ds(base, _BPW)], seq_all)

    def pair(j, _):
        for k, buf, sem in ((0, buf0, sem0), (1, buf1, sem1)):
            i = 2 * j + k

            @pl.when(j > 0)
            def _():
                pltpu.make_async_copy(out_hbm.at[base], buf, sem).wait()

            _build_block(seq_all, tabt_v, buf, i)
            pltpu.async_copy(
                buf.at[:, pl.ds(0, 128)], out_hbm.at[base + i, :, pl.ds(0, 128)], sem
            )
            pltpu.async_copy(
                buf.at[:, pl.ds(128, 72)],
                out_hbm.at[base + i, :, pl.ds(128, 72)],
                sem,
            )
        return 0

    lax.fori_loop(0, _BPW // 2, pair, 0)
    pltpu.make_async_copy(out_hbm.at[base], buf0, sem0).wait()
    pltpu.make_async_copy(out_hbm.at[base], buf1, sem1).wait()


def kernel(seq, table):
    seq = seq.astype(jnp.int32)
    tabT16 = jnp.zeros((_D, 16), jnp.float32).at[:, :_V].set(table.T)
    mesh = plsc.VectorSubcoreMesh(core_axis_name="c", subcore_axis_name="s")
    k = functools.partial(
        pl.kernel,
        mesh=mesh,
        out_type=jax.ShapeDtypeStruct((_B, _D, _L), jnp.float32),
        scratch_types=[
            pltpu.VMEM((_BPW, _L), jnp.int32),
            pltpu.VMEM((_D, 16), jnp.float32),
            pltpu.VMEM((_D, _L), jnp.float32),
            pltpu.VMEM((_D, _L), jnp.float32),
            pltpu.SemaphoreType.DMA,
            pltpu.SemaphoreType.DMA,
        ],
        compiler_params=pltpu.CompilerParams(needs_layout_passes=False),
    )(_sc_body)
    return k(seq, tabT16)
